# Initial kernel scaffold; baseline (speedup 1.0000x reference)
import jax
import jax.numpy as jnp
from jax.experimental import pallas as pl


def kernel(ids_true_batch, ids_false_batch, ent_table):
    raise NotImplementedError("write your pallas kernel here")


# speedup vs baseline: 1.1011x; 1.1011x over previous
import jax
import jax.numpy as jnp
from jax import lax
from jax.experimental import pallas as pl
from jax.experimental.pallas import tpu as pltpu
from jax.experimental.pallas import tpu_sc as plsc

EMB_DIM = 64
BATCH = 16384
NC = 2
NS = 16
NW = NC * NS
BPW = BATCH // NW
NIDS = 5
CHUNK = 128
NCHUNK = NIDS * BPW // CHUNK
GROUPS = CHUNK // 16


def _rsqrt16(x):
    half = jnp.full((16,), 0.5, jnp.float32)
    three_half = jnp.full((16,), 1.5, jnp.float32)
    i = plsc.bitcast(x, jnp.int32)
    i = jnp.full((16,), 0x5F3759DF, jnp.int32) - (i >> 1)
    y = plsc.bitcast(i, jnp.float32)
    for _ in range(3):
        y = y * (three_half - half * x * y * y)
    return y


def _body(idx_hbm, table_hbm, out_t, out_hf, out_tf,
          idx_v, rows_v, s_v, ot_v, ohf_v, otf_v, sem):
    w = lax.axis_index("s") * NC + lax.axis_index("c")
    pltpu.sync_copy(idx_hbm.at[w], idx_v)

    lane = lax.iota(jnp.int32, 16)

    def chunk_body(c, carry):
        pltpu.async_copy(table_hbm.at[idx_v.at[c]], rows_v, sem).wait()

        def group_body(t, carry2):
            rows16 = t * 16 + lane
            acc = jnp.zeros((16,), jnp.float32)
            acc2 = jnp.zeros((16,), jnp.float32)
            for j in range(EMB_DIM):
                col = jnp.full((16,), j, jnp.int32)
                x = plsc.load_gather(rows_v, [rows16, col])
                acc = acc + x
                acc2 = acc2 + x * x
            s_v[pl.ds(c * CHUNK + t * 16, 16)] = acc * _rsqrt16(acc2)
            return carry2

        lax.fori_loop(0, GROUPS, group_body, 0)
        return carry

    lax.fori_loop(0, NCHUNK, chunk_body, 0)

    def comb_body(i, carry):
        o = i * 16
        sh = s_v[pl.ds(o, 16)]
        sr = s_v[pl.ds(BPW + o, 16)]
        st = s_v[pl.ds(2 * BPW + o, 16)]
        shf = s_v[pl.ds(3 * BPW + o, 16)]
        stf = s_v[pl.ds(4 * BPW + o, 16)]
        ot_v[pl.ds(o, 16)] = sh + sr - st
        ohf_v[pl.ds(o, 16)] = shf + sr - st
        otf_v[pl.ds(o, 16)] = sh + sr - stf
        return carry

    lax.fori_loop(0, BPW // 16, comb_body, 0)

    base = w * BPW
    pltpu.sync_copy(ot_v, out_t.at[pl.ds(base, BPW)])
    pltpu.sync_copy(ohf_v, out_hf.at[pl.ds(base, BPW)])
    pltpu.sync_copy(otf_v, out_tf.at[pl.ds(base, BPW)])


def kernel(ids_true_batch, ids_false_batch, ent_table):
    ids_all = jnp.concatenate([ids_true_batch, ids_false_batch], axis=0)
    ids_all = ids_all.astype(jnp.int32)
    idx = ids_all.reshape(NIDS, NW, BPW).transpose(1, 0, 2)
    idx = idx.reshape(NW, NCHUNK, CHUNK)

    mesh = plsc.VectorSubcoreMesh(core_axis_name="c", subcore_axis_name="s")
    fn = pl.kernel(
        _body,
        mesh=mesh,
        compiler_params=pltpu.CompilerParams(
            needs_layout_passes=False, use_tc_tiling_on_sc=False
        ),
        out_type=[jax.ShapeDtypeStruct((BATCH,), jnp.float32)] * 3,
        scratch_types=[
            pltpu.VMEM((NCHUNK, CHUNK), jnp.int32),
            pltpu.VMEM((CHUNK, EMB_DIM), jnp.float32),
            pltpu.VMEM((NIDS * BPW,), jnp.float32),
            pltpu.VMEM((BPW,), jnp.float32),
            pltpu.VMEM((BPW,), jnp.float32),
            pltpu.VMEM((BPW,), jnp.float32),
            pltpu.SemaphoreType.DMA,
        ],
    )
    t, hf, tf = fn(idx, ent_table)
    return (t, hf, tf)
